# Initial kernel scaffold; baseline (speedup 1.0000x reference)
import jax
import jax.numpy as jnp
from jax.experimental import pallas as pl


def kernel(x_user, x_item, params, edge_index_user_item, edge_index_item_user):
    raise NotImplementedError("write your pallas kernel here")


# speedup vs baseline: 4.3654x; 4.3654x over previous
import functools

import jax
import jax.numpy as jnp
from jax import lax
from jax.experimental import pallas as pl
from jax.experimental.pallas import tpu as pltpu
from jax.experimental.pallas import tpu_sc as plsc

N_NODE = 10000
D = 128
E_EDGE = 160000
NEG = 0.2
NUM_LAYERS = 2
EPS = 1e-16

NTILE = 16
CHUNK = 128
EPAD = 163840
CPT = EPAD // (NTILE * CHUNK)
PAD_DST = 10200

ACC_ROWS = 10240
HALF = ACC_ROWS // 2
SPM_ROWS = 5248
MSG_SCRAP = 5120
DEN0 = 5128
DEN_ROWS = ACC_ROWS // 128
ZPT = SPM_ROWS // NTILE
WPT = HALF // NTILE

EB = 2048


def _proj_body(x_ref, w1_ref, w2_ref, o1_ref, o2_ref):
    x = x_ref[...]
    o1_ref[...] = jnp.dot(x, w1_ref[...], preferred_element_type=jnp.float32)
    o2_ref[...] = jnp.dot(x, w2_ref[...], preferred_element_type=jnp.float32)


def _proj2(x, w1, w2):
    br = 1000
    return pl.pallas_call(
        _proj_body,
        grid=(N_NODE // br,),
        in_specs=[
            pl.BlockSpec((br, D), lambda i: (i, 0)),
            pl.BlockSpec((D, D), lambda i: (0, 0)),
            pl.BlockSpec((D, D), lambda i: (0, 0)),
        ],
        out_specs=[pl.BlockSpec((br, D), lambda i: (i, 0))] * 2,
        out_shape=[jax.ShapeDtypeStruct((N_NODE, D), jnp.float32)] * 2,
    )(x, w1, w2)


def _edge_body(xlg_ref, xrg_ref, att_ref, dcol_ref, msg_ref, den_ref):
    xl = xlg_ref[...]
    e = xl + xrg_ref[...]
    e = jnp.where(e > 0, e, NEG * e)
    attrow = att_ref[0:1, :]
    ev = jnp.exp(jnp.sum(e * attrow, axis=1, keepdims=True))
    msg_ref[...] = xl * ev
    dmod = dcol_ref[...] % 128
    oh = lax.broadcasted_iota(jnp.int32, (EB, D), 1) == dmod
    den_ref[...] = jnp.where(oh, ev, 0.0)


def _edge_tc(xlg, xrg, att8, dcol):
    return pl.pallas_call(
        _edge_body,
        grid=(EPAD // EB,),
        in_specs=[
            pl.BlockSpec((EB, D), lambda i: (i, 0)),
            pl.BlockSpec((EB, D), lambda i: (i, 0)),
            pl.BlockSpec((8, D), lambda i: (0, 0)),
            pl.BlockSpec((EB, 1), lambda i: (i, 0)),
        ],
        out_specs=[pl.BlockSpec((EB, D), lambda i: (i, 0))] * 2,
        out_shape=[jax.ShapeDtypeStruct((EPAD, D), jnp.float32)] * 2,
    )(xlg, xrg, att8, dcol)


def _fin_body(acc_ref, den_ref, b_ref, o_ref):
    rv = 1.0 / (den_ref[...] + EPS)
    o_ref[...] = acc_ref[...] * rv + b_ref[0:1, :]


def _fin_tc(acc, dencol, b8):
    br = 1000
    return pl.pallas_call(
        _fin_body,
        grid=(N_NODE // br,),
        in_specs=[
            pl.BlockSpec((br, D), lambda i: (i, 0)),
            pl.BlockSpec((br, 1), lambda i: (i, 0)),
            pl.BlockSpec((8, D), lambda i: (0, 0)),
        ],
        out_specs=pl.BlockSpec((br, D), lambda i: (i, 0)),
        out_shape=jax.ShapeDtypeStruct((N_NODE, D), jnp.float32),
    )(acc, dencol, b8)


def _gather_conv(s, xl_h, xr_h, si_h, di_h, xlg_h, xrg_h,
                 sidx, didx, xlbuf, xrbuf, sem0, sem1):
    pltpu.sync_copy(si_h.at[pl.ds(s * CPT, CPT)], sidx)
    pltpu.sync_copy(di_h.at[pl.ds(s * CPT, CPT)], didx)

    def chunk(g, carry):
        cp0 = pltpu.async_copy(xl_h.at[sidx.at[g]], xlbuf, sem0)
        cp1 = pltpu.async_copy(xr_h.at[didx.at[g]], xrbuf, sem1)
        cp0.wait()
        cp1.wait()
        base = (s * CPT + g) * CHUNK
        pltpu.sync_copy(xlbuf, xlg_h.at[pl.ds(base, CHUNK)])
        pltpu.sync_copy(xrbuf, xrg_h.at[pl.ds(base, CHUNK)])
        return carry

    lax.fori_loop(0, CPT, chunk, 0)


@functools.partial(
    pl.kernel,
    mesh=plsc.VectorSubcoreMesh(core_axis_name="c", subcore_axis_name="s"),
    out_type=tuple(jax.ShapeDtypeStruct((EPAD, D), jnp.float32)
                   for _ in range(4)),
    scratch_types=[
        pltpu.VMEM((CPT, CHUNK), jnp.int32),
        pltpu.VMEM((CPT, CHUNK), jnp.int32),
        pltpu.VMEM((CHUNK, D), jnp.float32),
        pltpu.VMEM((CHUNK, D), jnp.float32),
        pltpu.SemaphoreType.DMA,
        pltpu.SemaphoreType.DMA,
    ],
)
def _sc_gather(xlu, xri, si_ui, di_ui, xli, xru, si_iu, di_iu,
               xlg_ui, xrg_ui, xlg_iu, xrg_iu,
               sidx, didx, xlbuf, xrbuf, sem0, sem1):
    c = lax.axis_index("c")
    s = lax.axis_index("s")

    @pl.when(c == 0)
    def _():
        _gather_conv(s, xlu, xri, si_ui, di_ui, xlg_ui, xrg_ui,
                     sidx, didx, xlbuf, xrbuf, sem0, sem1)

    @pl.when(c == 1)
    def _():
        _gather_conv(s, xli, xru, si_iu, di_iu, xlg_iu, xrg_iu,
                     sidx, didx, xlbuf, xrbuf, sem0, sem1)


def _zero_spm(s, msgbuf, spm):
    zero16 = jnp.zeros((16,), jnp.float32)

    def zrow(r, carry):
        for j in range(D // 16):
            msgbuf[r, pl.ds(j * 16, 16)] = zero16
        return carry

    lax.fori_loop(0, CHUNK, zrow, 0)
    zbase = s * ZPT
    pltpu.sync_copy(msgbuf, spm.at[pl.ds(zbase, CHUNK)])
    pltpu.sync_copy(msgbuf, spm.at[pl.ds(zbase + CHUNK, CHUNK)])
    pltpu.sync_copy(msgbuf.at[pl.ds(0, ZPT - 2 * CHUNK)],
                    spm.at[pl.ds(zbase + 2 * CHUNK, ZPT - 2 * CHUNK)])


def _scatter_conv(s, msg_h, den_h, di_h, acc_h, denout_h,
                  didx, msgbuf, denbuf, ria, drid, spm):
    _zero_spm(s, msgbuf, spm)
    pltpu.sync_copy(di_h.at[pl.ds(s * CPT, CPT)], didx)
    plsc.subcore_barrier()

    def chunk_a(g, carry):
        base = (s * CPT + g) * CHUNK
        pltpu.sync_copy(msg_h.at[pl.ds(base, CHUNK)], msgbuf)
        pltpu.sync_copy(den_h.at[pl.ds(base, CHUNK)], denbuf)
        for j in range(CHUNK // 16):
            dv = didx[g, pl.ds(j * 16, 16)]
            ria[pl.ds(j * 16, 16)] = jnp.where(dv < HALF, dv, MSG_SCRAP)
            drid[pl.ds(j * 16, 16)] = DEN0 + lax.shift_right_logical(dv, 7)
        pltpu.sync_copy(msgbuf, spm.at[ria], add=True)
        pltpu.sync_copy(denbuf, spm.at[drid], add=True)
        return carry

    lax.fori_loop(0, CPT, chunk_a, 0)
    plsc.subcore_barrier()

    pltpu.sync_copy(spm.at[pl.ds(s * WPT, WPT)], acc_h.at[pl.ds(s * WPT, WPT)])

    @pl.when(s == 0)
    def _():
        pltpu.sync_copy(spm.at[pl.ds(DEN0, DEN_ROWS)], denout_h)

    plsc.subcore_barrier()

    _zero_spm(s, msgbuf, spm)
    plsc.subcore_barrier()

    def chunk_b(g, carry):
        base = (s * CPT + g) * CHUNK
        pltpu.sync_copy(msg_h.at[pl.ds(base, CHUNK)], msgbuf)
        for j in range(CHUNK // 16):
            dv = didx[g, pl.ds(j * 16, 16)]
            ria[pl.ds(j * 16, 16)] = jnp.where(dv >= HALF, dv - HALF,
                                               MSG_SCRAP)
        pltpu.sync_copy(msgbuf, spm.at[ria], add=True)
        return carry

    lax.fori_loop(0, CPT, chunk_b, 0)
    plsc.subcore_barrier()

    pltpu.sync_copy(spm.at[pl.ds(s * WPT, WPT)],
                    acc_h.at[pl.ds(HALF + s * WPT, WPT)])


@functools.partial(
    pl.kernel,
    mesh=plsc.VectorSubcoreMesh(core_axis_name="c", subcore_axis_name="s"),
    out_type=(jax.ShapeDtypeStruct((ACC_ROWS, D), jnp.float32),
              jax.ShapeDtypeStruct((DEN_ROWS, D), jnp.float32),
              jax.ShapeDtypeStruct((ACC_ROWS, D), jnp.float32),
              jax.ShapeDtypeStruct((DEN_ROWS, D), jnp.float32)),
    scratch_types=[
        pltpu.VMEM((CPT, CHUNK), jnp.int32),
        pltpu.VMEM((CHUNK, D), jnp.float32),
        pltpu.VMEM((CHUNK, D), jnp.float32),
        pltpu.VMEM((CHUNK,), jnp.int32),
        pltpu.VMEM((CHUNK,), jnp.int32),
        pltpu.VMEM_SHARED((SPM_ROWS, D), jnp.float32),
    ],
)
def _sc_scatter(msg_ui, den_ui, di_ui, msg_iu, den_iu, di_iu,
                acc_ui, denout_ui, acc_iu, denout_iu,
                didx, msgbuf, denbuf, ria, drid, spm):
    c = lax.axis_index("c")
    s = lax.axis_index("s")

    @pl.when(c == 0)
    def _():
        _scatter_conv(s, msg_ui, den_ui, di_ui, acc_ui, denout_ui,
                      didx, msgbuf, denbuf, ria, drid, spm)

    @pl.when(c == 1)
    def _():
        _scatter_conv(s, msg_iu, den_iu, di_iu, acc_iu, denout_iu,
                      didx, msgbuf, denbuf, ria, drid, spm)


def _prep_edges(ei):
    src = ei[0].astype(jnp.int32)
    dst = ei[1].astype(jnp.int32)
    npad = EPAD - E_EDGE
    src = jnp.concatenate([src, jnp.zeros((npad,), jnp.int32)])
    dst = jnp.concatenate([dst, jnp.full((npad,), PAD_DST, jnp.int32)])
    return (src.reshape(EPAD // CHUNK, CHUNK),
            dst.reshape(EPAD // CHUNK, CHUNK),
            dst.reshape(EPAD, 1))


def kernel(x_user, x_item, params, edge_index_user_item, edge_index_item_user):
    si_ui, di_ui, dcol_ui = _prep_edges(edge_index_user_item)
    si_iu, di_iu, dcol_iu = _prep_edges(edge_index_item_user)

    def stack(et, name):
        arrs = [params["layer%d" % l][et][name] for l in range(NUM_LAYERS)]
        if name in ("att", "b"):
            arrs = [jnp.broadcast_to(a[None, :], (8, D)) for a in arrs]
        return jnp.stack(arrs)

    xs = tuple(stack(et, nm) for et in ("ui", "iu")
               for nm in ("Wl", "Wr", "att", "b"))

    def step(carry, wts):
        xu, xi = carry
        wl_ui, wr_ui, att_ui, b_ui, wl_iu, wr_iu, att_iu, b_iu = wts
        xlu, xru = _proj2(xu, wl_ui, wr_iu)
        xli, xri = _proj2(xi, wl_iu, wr_ui)
        xlg_ui, xrg_ui, xlg_iu, xrg_iu = _sc_gather(
            xlu, xri, si_ui, di_ui, xli, xru, si_iu, di_iu)
        msg_ui, dr_ui = _edge_tc(xlg_ui, xrg_ui, att_ui, dcol_ui)
        msg_iu, dr_iu = _edge_tc(xlg_iu, xrg_iu, att_iu, dcol_iu)
        acc_ui, den_ui, acc_iu, den_iu = _sc_scatter(
            msg_ui, dr_ui, di_ui, msg_iu, dr_iu, di_iu)
        dencol_i = den_ui.reshape(ACC_ROWS, 1)[:N_NODE]
        dencol_u = den_iu.reshape(ACC_ROWS, 1)[:N_NODE]
        xi_new = _fin_tc(acc_ui[:N_NODE], dencol_i, b_ui)
        xu_new = _fin_tc(acc_iu[:N_NODE], dencol_u, b_iu)
        return (xu_new, xi_new), None

    (xu, xi), _ = lax.scan(step, (x_user, x_item), xs)
    return (xu, xi)


# speedup vs baseline: 4.8021x; 1.1000x over previous
import functools

import jax
import jax.numpy as jnp
from jax import lax
from jax.experimental import pallas as pl
from jax.experimental.pallas import tpu as pltpu
from jax.experimental.pallas import tpu_sc as plsc

N_NODE = 10000
D = 128
E_EDGE = 160000
NEG = 0.2
NUM_LAYERS = 2
EPS = 1e-16

NTILE = 16
CHUNK = 128
EPAD = 163840
CPT = EPAD // (NTILE * CHUNK)
PAD_DST = 10200

ACC_ROWS = 10240
HALF = ACC_ROWS // 2
SPM_ROWS = 5248
MSG_SCRAP = 5120
DEN0 = 5128
DEN_ROWS = ACC_ROWS // 128
ZPT = SPM_ROWS // NTILE
WPT = HALF // NTILE

EB = 2048


def _proj_body(x_ref, w1_ref, w2_ref, o1_ref, o2_ref):
    x = x_ref[...]
    o1_ref[...] = jnp.dot(x, w1_ref[...], preferred_element_type=jnp.float32)
    o2_ref[...] = jnp.dot(x, w2_ref[...], preferred_element_type=jnp.float32)


def _proj2(x, w1, w2):
    br = 1000
    return pl.pallas_call(
        _proj_body,
        grid=(N_NODE // br,),
        in_specs=[
            pl.BlockSpec((br, D), lambda i: (i, 0)),
            pl.BlockSpec((D, D), lambda i: (0, 0)),
            pl.BlockSpec((D, D), lambda i: (0, 0)),
        ],
        out_specs=[pl.BlockSpec((br, D), lambda i: (i, 0))] * 2,
        out_shape=[jax.ShapeDtypeStruct((N_NODE, D), jnp.float32)] * 2,
    )(x, w1, w2)


def _edge_body(xlg_ref, xrg_ref, att_ref, dcol_ref, msg_ref, den_ref):
    xl = xlg_ref[...]
    e = xl + xrg_ref[...]
    e = jnp.where(e > 0, e, NEG * e)
    attrow = att_ref[0:1, :]
    ev = jnp.exp(jnp.sum(e * attrow, axis=1, keepdims=True))
    msg_ref[...] = xl * ev
    dmod = dcol_ref[...] % 128
    oh = lax.broadcasted_iota(jnp.int32, (EB, D), 1) == dmod
    den_ref[...] = jnp.where(oh, ev, 0.0)


def _edge_tc(xlg, xrg, att8, dcol):
    return pl.pallas_call(
        _edge_body,
        grid=(EPAD // EB,),
        in_specs=[
            pl.BlockSpec((EB, D), lambda i: (i, 0)),
            pl.BlockSpec((EB, D), lambda i: (i, 0)),
            pl.BlockSpec((8, D), lambda i: (0, 0)),
            pl.BlockSpec((EB, 1), lambda i: (i, 0)),
        ],
        out_specs=[pl.BlockSpec((EB, D), lambda i: (i, 0))] * 2,
        out_shape=[jax.ShapeDtypeStruct((EPAD, D), jnp.float32)] * 2,
    )(xlg, xrg, att8, dcol)


def _fin_body(acc_ref, den_ref, b_ref, o_ref):
    rv = 1.0 / (den_ref[...] + EPS)
    o_ref[...] = acc_ref[...] * rv + b_ref[0:1, :]


def _fin_tc(acc, dencol, b8):
    br = 1000
    return pl.pallas_call(
        _fin_body,
        grid=(N_NODE // br,),
        in_specs=[
            pl.BlockSpec((br, D), lambda i: (i, 0)),
            pl.BlockSpec((br, 1), lambda i: (i, 0)),
            pl.BlockSpec((8, D), lambda i: (0, 0)),
        ],
        out_specs=pl.BlockSpec((br, D), lambda i: (i, 0)),
        out_shape=jax.ShapeDtypeStruct((N_NODE, D), jnp.float32),
    )(acc, dencol, b8)


def _gather_conv(s, xl_h, xr_h, si_h, di_h, xlg_h, xrg_h,
                 sidx, didx, bufs, gsems, wsems):
    pltpu.sync_copy(si_h.at[pl.ds(s * CPT, CPT)], sidx)
    pltpu.sync_copy(di_h.at[pl.ds(s * CPT, CPT)], didx)

    def src(p, g):
        idx = sidx if p % 2 == 0 else didx
        tab = xl_h if p % 2 == 0 else xr_h
        return tab.at[idx.at[g]]

    def dst(p, g):
        out = xlg_h if p % 2 == 0 else xrg_h
        return out.at[pl.ds((s * CPT + g) * CHUNK, CHUNK)]

    for p in range(4):
        pltpu.async_copy(src(p, p // 2), bufs[p], gsems[p])

    def pair(h, carry):
        for q in range(2):
            g = 2 * h + q
            for p in (2 * q, 2 * q + 1):
                pltpu.make_async_copy(src(p, g), bufs[p], gsems[p]).wait()
                pltpu.async_copy(bufs[p], dst(p, g), wsems[p])
        for q in range(2):
            g = 2 * h + q
            for p in (2 * q, 2 * q + 1):
                pltpu.make_async_copy(bufs[p], dst(p, g), wsems[p]).wait()

                @pl.when(g + 2 < CPT)
                def _():
                    pltpu.async_copy(src(p, g + 2), bufs[p], gsems[p])
        return carry

    lax.fori_loop(0, CPT // 2, pair, 0)


@functools.partial(
    pl.kernel,
    mesh=plsc.VectorSubcoreMesh(core_axis_name="c", subcore_axis_name="s"),
    out_type=tuple(jax.ShapeDtypeStruct((EPAD, D), jnp.float32)
                   for _ in range(4)),
    scratch_types=[
        pltpu.VMEM((CPT, CHUNK), jnp.int32),
        pltpu.VMEM((CPT, CHUNK), jnp.int32),
        pltpu.VMEM((CHUNK, D), jnp.float32),
        pltpu.VMEM((CHUNK, D), jnp.float32),
        pltpu.VMEM((CHUNK, D), jnp.float32),
        pltpu.VMEM((CHUNK, D), jnp.float32),
    ] + [pltpu.SemaphoreType.DMA] * 8,
)
def _sc_gather(xlu, xri, si_ui, di_ui, xli, xru, si_iu, di_iu,
               xlg_ui, xrg_ui, xlg_iu, xrg_iu,
               sidx, didx, b0, b1, b2, b3,
               g0, g1, g2, g3, w0, w1, w2, w3):
    c = lax.axis_index("c")
    s = lax.axis_index("s")
    bufs = [b0, b1, b2, b3]
    gsems = [g0, g1, g2, g3]
    wsems = [w0, w1, w2, w3]

    @pl.when(c == 0)
    def _():
        _gather_conv(s, xlu, xri, si_ui, di_ui, xlg_ui, xrg_ui,
                     sidx, didx, bufs, gsems, wsems)

    @pl.when(c == 1)
    def _():
        _gather_conv(s, xli, xru, si_iu, di_iu, xlg_iu, xrg_iu,
                     sidx, didx, bufs, gsems, wsems)


def _zero_spm(s, msgbuf, spm):
    zero16 = jnp.zeros((16,), jnp.float32)

    def zrow(r, carry):
        for j in range(D // 16):
            msgbuf[r, pl.ds(j * 16, 16)] = zero16
        return carry

    lax.fori_loop(0, CHUNK, zrow, 0)
    zbase = s * ZPT
    pltpu.sync_copy(msgbuf, spm.at[pl.ds(zbase, CHUNK)])
    pltpu.sync_copy(msgbuf, spm.at[pl.ds(zbase + CHUNK, CHUNK)])
    pltpu.sync_copy(msgbuf.at[pl.ds(0, ZPT - 2 * CHUNK)],
                    spm.at[pl.ds(zbase + 2 * CHUNK, ZPT - 2 * CHUNK)])


def _scatter_pass(s, src_h, di_h, didx, bufs, ids, lsems, ssems, spm,
                  remap, with_ids):
    def src(g):
        return src_h.at[pl.ds((s * CPT + g) * CHUNK, CHUNK)]

    for q in range(2):
        pltpu.async_copy(src(q), bufs[q], lsems[q])

    def pair(h, carry):
        for q in range(2):
            g = 2 * h + q
            pltpu.make_async_copy(src(g), bufs[q], lsems[q]).wait()
            if with_ids:
                for j in range(CHUNK // 16):
                    dv = didx[g, pl.ds(j * 16, 16)]
                    ids[q][pl.ds(j * 16, 16)] = remap(dv)
            pltpu.async_copy(bufs[q], spm.at[ids[q]], ssems[q], add=True)
        for q in range(2):
            g = 2 * h + q
            pltpu.make_async_copy(bufs[q], spm.at[ids[q]], ssems[q]).wait()

            @pl.when(g + 2 < CPT)
            def _():
                pltpu.async_copy(src(g + 2), bufs[q], lsems[q])
        return carry

    lax.fori_loop(0, CPT // 2, pair, 0)


def _scatter_conv(s, msg_h, den_h, di_h, acc_h, denout_h,
                  didx, m0, m1, d0, d1, ria0, ria1, drid0, drid1,
                  lm0, lm1, ld0, ld1, sm0, sm1, sd0, sd1, spm):
    _zero_spm(s, m0, spm)
    pltpu.sync_copy(di_h.at[pl.ds(s * CPT, CPT)], didx)
    plsc.subcore_barrier()

    def remap_a(dv):
        return jnp.where(dv < HALF, dv, MSG_SCRAP)

    def remap_den(dv):
        return DEN0 + lax.shift_right_logical(dv, 7)

    _scatter_pass(s, msg_h, di_h, didx, [m0, m1], [ria0, ria1],
                  [lm0, lm1], [sm0, sm1], spm, remap_a, True)
    _scatter_pass(s, den_h, di_h, didx, [d0, d1], [drid0, drid1],
                  [ld0, ld1], [sd0, sd1], spm, remap_den, True)
    plsc.subcore_barrier()

    pltpu.sync_copy(spm.at[pl.ds(s * WPT, WPT)], acc_h.at[pl.ds(s * WPT, WPT)])

    @pl.when(s == 0)
    def _():
        pltpu.sync_copy(spm.at[pl.ds(DEN0, DEN_ROWS)], denout_h)

    plsc.subcore_barrier()

    _zero_spm(s, m0, spm)
    plsc.subcore_barrier()

    def remap_b(dv):
        return jnp.where(dv >= HALF, dv - HALF, MSG_SCRAP)

    _scatter_pass(s, msg_h, di_h, didx, [m0, m1], [ria0, ria1],
                  [lm0, lm1], [sm0, sm1], spm, remap_b, True)
    plsc.subcore_barrier()

    pltpu.sync_copy(spm.at[pl.ds(s * WPT, WPT)],
                    acc_h.at[pl.ds(HALF + s * WPT, WPT)])


@functools.partial(
    pl.kernel,
    mesh=plsc.VectorSubcoreMesh(core_axis_name="c", subcore_axis_name="s"),
    out_type=(jax.ShapeDtypeStruct((ACC_ROWS, D), jnp.float32),
              jax.ShapeDtypeStruct((DEN_ROWS, D), jnp.float32),
              jax.ShapeDtypeStruct((ACC_ROWS, D), jnp.float32),
              jax.ShapeDtypeStruct((DEN_ROWS, D), jnp.float32)),
    scratch_types=[
        pltpu.VMEM((CPT, CHUNK), jnp.int32),
        pltpu.VMEM((CHUNK, D), jnp.float32),
        pltpu.VMEM((CHUNK, D), jnp.float32),
        pltpu.VMEM((CHUNK, D), jnp.float32),
        pltpu.VMEM((CHUNK, D), jnp.float32),
        pltpu.VMEM((CHUNK,), jnp.int32),
        pltpu.VMEM((CHUNK,), jnp.int32),
        pltpu.VMEM((CHUNK,), jnp.int32),
        pltpu.VMEM((CHUNK,), jnp.int32),
    ] + [pltpu.SemaphoreType.DMA] * 8 + [
        pltpu.VMEM_SHARED((SPM_ROWS, D), jnp.float32),
    ],
)
def _sc_scatter(msg_ui, den_ui, di_ui, msg_iu, den_iu, di_iu,
                acc_ui, denout_ui, acc_iu, denout_iu,
                didx, m0, m1, d0, d1, ria0, ria1, drid0, drid1,
                lm0, lm1, ld0, ld1, sm0, sm1, sd0, sd1, spm):
    c = lax.axis_index("c")
    s = lax.axis_index("s")

    @pl.when(c == 0)
    def _():
        _scatter_conv(s, msg_ui, den_ui, di_ui, acc_ui, denout_ui,
                      didx, m0, m1, d0, d1, ria0, ria1, drid0, drid1,
                      lm0, lm1, ld0, ld1, sm0, sm1, sd0, sd1, spm)

    @pl.when(c == 1)
    def _():
        _scatter_conv(s, msg_iu, den_iu, di_iu, acc_iu, denout_iu,
                      didx, m0, m1, d0, d1, ria0, ria1, drid0, drid1,
                      lm0, lm1, ld0, ld1, sm0, sm1, sd0, sd1, spm)


def _prep_edges(ei):
    src = ei[0].astype(jnp.int32)
    dst = ei[1].astype(jnp.int32)
    npad = EPAD - E_EDGE
    src = jnp.concatenate([src, jnp.zeros((npad,), jnp.int32)])
    dst = jnp.concatenate([dst, jnp.full((npad,), PAD_DST, jnp.int32)])
    return (src.reshape(EPAD // CHUNK, CHUNK),
            dst.reshape(EPAD // CHUNK, CHUNK),
            dst.reshape(EPAD, 1))


def kernel(x_user, x_item, params, edge_index_user_item, edge_index_item_user):
    si_ui, di_ui, dcol_ui = _prep_edges(edge_index_user_item)
    si_iu, di_iu, dcol_iu = _prep_edges(edge_index_item_user)

    def stack(et, name):
        arrs = [params["layer%d" % l][et][name] for l in range(NUM_LAYERS)]
        if name in ("att", "b"):
            arrs = [jnp.broadcast_to(a[None, :], (8, D)) for a in arrs]
        return jnp.stack(arrs)

    xs = tuple(stack(et, nm) for et in ("ui", "iu")
               for nm in ("Wl", "Wr", "att", "b"))

    def step(carry, wts):
        xu, xi = carry
        wl_ui, wr_ui, att_ui, b_ui, wl_iu, wr_iu, att_iu, b_iu = wts
        xlu, xru = _proj2(xu, wl_ui, wr_iu)
        xli, xri = _proj2(xi, wl_iu, wr_ui)
        xlg_ui, xrg_ui, xlg_iu, xrg_iu = _sc_gather(
            xlu, xri, si_ui, di_ui, xli, xru, si_iu, di_iu)
        msg_ui, dr_ui = _edge_tc(xlg_ui, xrg_ui, att_ui, dcol_ui)
        msg_iu, dr_iu = _edge_tc(xlg_iu, xrg_iu, att_iu, dcol_iu)
        acc_ui, den_ui, acc_iu, den_iu = _sc_scatter(
            msg_ui, dr_ui, di_ui, msg_iu, dr_iu, di_iu)
        dencol_i = den_ui.reshape(ACC_ROWS, 1)[:N_NODE]
        dencol_u = den_iu.reshape(ACC_ROWS, 1)[:N_NODE]
        xi_new = _fin_tc(acc_ui[:N_NODE], dencol_i, b_ui)
        xu_new = _fin_tc(acc_iu[:N_NODE], dencol_u, b_iu)
        return (xu_new, xi_new), None

    (xu, xi), _ = lax.scan(step, (x_user, x_item), xs)
    return (xu, xi)
